# baseline (speedup 1.0000x reference)
import functools

import jax
import jax.numpy as jnp
from jax import lax
from jax.experimental import pallas as pl
from jax.experimental.pallas import tpu as pltpu
from jax.experimental.pallas import tpu_sc as plsc

N = 10000
H = 128
OUT = 16
R = 3
E = 200000

NC = 2
NS = 16
TILES = NC * NS

K = 128
NBUF = 3
CPT = 51
E_PAD = CPT * TILES * K
PAD = E_PAD - E

WB = 80
NWB = N // WB
WPT = -(-NWB // NS)


def _make_sc_agg(h_dim):
    mesh = plsc.VectorSubcoreMesh(core_axis_name="c", subcore_axis_name="s")

    @functools.partial(
        pl.kernel,
        mesh=mesh,
        out_type=jax.ShapeDtypeStruct((NC, N, h_dim), jnp.float32),
        scratch_types=[
            pltpu.VMEM_SHARED((N + 8, h_dim), jnp.float32),
            pltpu.VMEM((NBUF, 2, K), jnp.int32),
            pltpu.VMEM((K, h_dim), jnp.float32),
            pltpu.VMEM((K, h_dim), jnp.float32),
            pltpu.VMEM((K, h_dim), jnp.float32),
            pltpu.SemaphoreType.DMA,
            pltpu.SemaphoreType.DMA,
            pltpu.SemaphoreType.DMA,
        ],
    )
    def agg(t0, t1, t2, eall, zeros, out, acc, eidx, rows0, rows1, rows2,
            gs0, gs1, gs2):
        c = lax.axis_index("c")
        s = lax.axis_index("s")
        wid = c * NS + s
        bufs = ((rows0, gs0), (rows1, gs1), (rows2, gs2))

        pltpu.sync_copy(zeros, rows0.at[pl.ds(0, WB)])
        for k in range(WPT):
            m = s + NS * k

            @pl.when(m < NWB)
            def _(m=m):
                r0 = pl.multiple_of(m * WB, WB)
                pltpu.sync_copy(rows0.at[pl.ds(0, WB)], acc.at[pl.ds(r0, WB)])

        plsc.subcore_barrier()

        base0 = wid * CPT
        for r, tab in enumerate((t0, t1, t2)):
            roff = r * 2 * E_PAD

            def idx_copy(j, b, roff=roff):
                off = roff + (base0 + j) * K
                pltpu.sync_copy(eall.at[pl.ds(off, K)], eidx.at[b, 0])
                pltpu.sync_copy(eall.at[pl.ds(E_PAD + off, K)], eidx.at[b, 1])

            def gather(b, rb, sem, tab=tab):
                pltpu.async_copy(tab.at[eidx.at[b, 0]], rb, sem)

            def gather_wait(b, rb, sem, tab=tab):
                pltpu.make_async_copy(tab.at[eidx.at[b, 0]], rb, sem).wait()

            def scatter(b, rb):
                pltpu.sync_copy(rb, acc.at[eidx.at[b, 1]], add=True)

            for b, (rb, sem) in enumerate(bufs):
                idx_copy(b, b)
                gather(b, rb, sem)

            def step(t, _):
                for b, (rb, sem) in enumerate(bufs):
                    gather_wait(b, rb, sem)
                    scatter(b, rb)
                    idx_copy(NBUF * t + NBUF + b, b)
                    gather(b, rb, sem)
                return 0

            lax.fori_loop(0, CPT // NBUF - 1, step, 0)

            for b, (rb, sem) in enumerate(bufs):
                gather_wait(b, rb, sem)
                scatter(b, rb)

        plsc.subcore_barrier()

        for k in range(WPT):
            m = s + NS * k

            @pl.when(m < NWB)
            def _(m=m):
                r0 = pl.multiple_of(m * WB, WB)
                pltpu.sync_copy(acc.at[pl.ds(r0, WB)], rows0.at[pl.ds(0, WB)])
                pltpu.sync_copy(rows0.at[pl.ds(0, WB)], out.at[c, pl.ds(r0, WB)])

    return agg


_sc_agg_h = _make_sc_agg(H)


BN = 400


def _tc_dense_body(p_ref, b_ref, w_ref, o_ref):
    h = jnp.maximum(p_ref[0] + p_ref[1] + b_ref[0], 0.0)
    for r in range(R):
        o_ref[r] = jnp.dot(h, w_ref[r], preferred_element_type=jnp.float32)


def _tc_dense(part, b, w):
    return pl.pallas_call(
        _tc_dense_body,
        grid=(N // BN,),
        in_specs=[
            pl.BlockSpec((NC, BN, H), lambda i: (0, i, 0)),
            pl.BlockSpec((1, H), lambda i: (0, 0)),
            pl.BlockSpec((R, H, H), lambda i: (0, 0, 0)),
        ],
        out_specs=pl.BlockSpec((R, BN, H), lambda i: (0, i, 0)),
        out_shape=jax.ShapeDtypeStruct((R, N, H), jnp.float32),
    )(part, b, w)


def _tc_final_body(p_ref, b_ref, o_ref):
    o_ref[...] = p_ref[0, :, :OUT] + p_ref[1, :, :OUT] + b_ref[0]


def _tc_final(part, b):
    return pl.pallas_call(
        _tc_final_body,
        grid=(N // BN,),
        in_specs=[
            pl.BlockSpec((NC, BN, H), lambda i: (0, i, 0)),
            pl.BlockSpec((1, OUT), lambda i: (0, 0)),
        ],
        out_specs=pl.BlockSpec((BN, OUT), lambda i: (i, 0)),
        out_shape=jax.ShapeDtypeStruct((N, OUT), jnp.float32),
    )(part, b)


@jax.jit
def kernel(embed, b0, w1, b1, w2, b2, edge_index_0, edge_index_1,
           edge_index_2):
    zeros_h = jnp.zeros((WB, H), jnp.float32)
    w2_pad = jnp.zeros((R, H, H), jnp.float32).at[:, :, :OUT].set(w2)
    zpad = jnp.arange(PAD, dtype=jnp.int32) % N
    npad = N + (jnp.arange(PAD, dtype=jnp.int32) % 8)
    eall = jnp.concatenate(
        [jnp.concatenate([e[0], zpad, e[1], npad])
         for e in (edge_index_0, edge_index_1, edge_index_2)])

    p0 = _sc_agg_h(embed, embed, embed, eall, zeros_h)
    y = _tc_dense(p0, b0.reshape(1, H), w1)
    p1 = _sc_agg_h(y[0], y[1], y[2], eall, zeros_h)
    z = _tc_dense(p1, b1.reshape(1, H), w2_pad)
    p2 = _sc_agg_h(z[0], z[1], z[2], eall, zeros_h)
    return _tc_final(p2, b2.reshape(1, OUT))


# speedup vs baseline: 1.0728x; 1.0728x over previous
import functools

import jax
import jax.numpy as jnp
from jax import lax
from jax.experimental import pallas as pl
from jax.experimental.pallas import tpu as pltpu
from jax.experimental.pallas import tpu_sc as plsc

N = 10000
H = 128
OUT = 16
R = 3
E = 200000

NC = 2
NS = 16
TILES = NC * NS

K = 128
NBUF = 3
CPT = 51
E_PAD = CPT * TILES * K
PAD = E_PAD - E

WB = 80
NWB = N // WB
WPT = -(-NWB // NS)


def _make_sc_agg(h_dim):
    mesh = plsc.VectorSubcoreMesh(core_axis_name="c", subcore_axis_name="s")

    @functools.partial(
        pl.kernel,
        mesh=mesh,
        out_type=jax.ShapeDtypeStruct((NC, N, h_dim), jnp.float32),
        scratch_types=[
            pltpu.VMEM_SHARED((N + 8, h_dim), jnp.float32),
            pltpu.VMEM((NBUF, 2, K), jnp.int32),
            pltpu.VMEM((K, h_dim), jnp.float32),
            pltpu.VMEM((K, h_dim), jnp.float32),
            pltpu.VMEM((K, h_dim), jnp.float32),
            pltpu.SemaphoreType.DMA,
            pltpu.SemaphoreType.DMA,
            pltpu.SemaphoreType.DMA,
            pltpu.SemaphoreType.DMA,
            pltpu.SemaphoreType.DMA,
            pltpu.SemaphoreType.DMA,
        ],
    )
    def agg(t0, t1, t2, eall, zeros, out, acc, eidx, rows0, rows1, rows2,
            gs0, gs1, gs2, ss0, ss1, ss2):
        c = lax.axis_index("c")
        s = lax.axis_index("s")
        wid = c * NS + s
        bufs = ((rows0, gs0, ss0), (rows1, gs1, ss1), (rows2, gs2, ss2))

        pltpu.sync_copy(zeros, rows0.at[pl.ds(0, WB)])
        for k in range(WPT):
            m = s + NS * k

            @pl.when(m < NWB)
            def _(m=m):
                r0 = pl.multiple_of(m * WB, WB)
                pltpu.sync_copy(rows0.at[pl.ds(0, WB)], acc.at[pl.ds(r0, WB)])

        plsc.subcore_barrier()

        base0 = wid * CPT
        for r, tab in enumerate((t0, t1, t2)):
            roff = r * 2 * E_PAD

            def idx_copy(j, b, roff=roff):
                off = roff + (base0 + j) * K
                pltpu.sync_copy(eall.at[pl.ds(off, K)], eidx.at[b, 0])
                pltpu.sync_copy(eall.at[pl.ds(E_PAD + off, K)], eidx.at[b, 1])

            def gather(b, rb, sem, tab=tab):
                pltpu.async_copy(tab.at[eidx.at[b, 0]], rb, sem)

            def gather_wait(b, rb, sem, tab=tab):
                pltpu.make_async_copy(tab.at[eidx.at[b, 0]], rb, sem).wait()

            def scatter(b, rb, sem):
                pltpu.async_copy(rb, acc.at[eidx.at[b, 1]], sem, add=True)

            def scatter_wait(b, rb, sem):
                pltpu.make_async_copy(rb, acc.at[eidx.at[b, 1]], sem).wait()

            for b, (rb, gsem, ssem) in enumerate(bufs):
                idx_copy(b, b)
                gather(b, rb, gsem)

            def step(t, _):
                for b, (rb, gsem, ssem) in enumerate(bufs):
                    gather_wait(b, rb, gsem)
                    scatter(b, rb, ssem)
                for b, (rb, gsem, ssem) in enumerate(bufs):
                    scatter_wait(b, rb, ssem)
                    idx_copy(NBUF * t + NBUF + b, b)
                    gather(b, rb, gsem)
                return 0

            lax.fori_loop(0, CPT // NBUF - 1, step, 0)

            for b, (rb, gsem, ssem) in enumerate(bufs):
                gather_wait(b, rb, gsem)
                scatter(b, rb, ssem)
            for b, (rb, gsem, ssem) in enumerate(bufs):
                scatter_wait(b, rb, ssem)

        plsc.subcore_barrier()

        for k in range(WPT):
            m = s + NS * k

            @pl.when(m < NWB)
            def _(m=m):
                r0 = pl.multiple_of(m * WB, WB)
                pltpu.sync_copy(acc.at[pl.ds(r0, WB)], rows0.at[pl.ds(0, WB)])
                pltpu.sync_copy(rows0.at[pl.ds(0, WB)], out.at[c, pl.ds(r0, WB)])

    return agg


_sc_agg_h = _make_sc_agg(H)


BN = 400


def _tc_dense_body(p_ref, b_ref, w_ref, o_ref):
    h = jnp.maximum(p_ref[0] + p_ref[1] + b_ref[0], 0.0)
    for r in range(R):
        o_ref[r] = jnp.dot(h, w_ref[r], preferred_element_type=jnp.float32)


def _tc_dense(part, b, w):
    return pl.pallas_call(
        _tc_dense_body,
        grid=(N // BN,),
        in_specs=[
            pl.BlockSpec((NC, BN, H), lambda i: (0, i, 0)),
            pl.BlockSpec((1, H), lambda i: (0, 0)),
            pl.BlockSpec((R, H, H), lambda i: (0, 0, 0)),
        ],
        out_specs=pl.BlockSpec((R, BN, H), lambda i: (0, i, 0)),
        out_shape=jax.ShapeDtypeStruct((R, N, H), jnp.float32),
    )(part, b, w)


def _tc_final_body(p_ref, b_ref, o_ref):
    o_ref[...] = p_ref[0, :, :OUT] + p_ref[1, :, :OUT] + b_ref[0]


def _tc_final(part, b):
    return pl.pallas_call(
        _tc_final_body,
        grid=(N // BN,),
        in_specs=[
            pl.BlockSpec((NC, BN, H), lambda i: (0, i, 0)),
            pl.BlockSpec((1, OUT), lambda i: (0, 0)),
        ],
        out_specs=pl.BlockSpec((BN, OUT), lambda i: (i, 0)),
        out_shape=jax.ShapeDtypeStruct((N, OUT), jnp.float32),
    )(part, b)


@jax.jit
def kernel(embed, b0, w1, b1, w2, b2, edge_index_0, edge_index_1,
           edge_index_2):
    zeros_h = jnp.zeros((WB, H), jnp.float32)
    w2_pad = jnp.zeros((R, H, H), jnp.float32).at[:, :, :OUT].set(w2)
    zpad = jnp.arange(PAD, dtype=jnp.int32) % N
    npad = N + (jnp.arange(PAD, dtype=jnp.int32) % 8)
    eall = jnp.concatenate(
        [jnp.concatenate([e[0], zpad, e[1], npad])
         for e in (edge_index_0, edge_index_1, edge_index_2)])

    p0 = _sc_agg_h(embed, embed, embed, eall, zeros_h)
    y = _tc_dense(p0, b0.reshape(1, H), w1)
    p1 = _sc_agg_h(y[0], y[1], y[2], eall, zeros_h)
    z = _tc_dense(p1, b1.reshape(1, H), w2_pad)
    p2 = _sc_agg_h(z[0], z[1], z[2], eall, zeros_h)
    return _tc_final(p2, b2.reshape(1, OUT))


# speedup vs baseline: 1.0985x; 1.0239x over previous
import functools

import jax
import jax.numpy as jnp
from jax import lax
from jax.experimental import pallas as pl
from jax.experimental.pallas import tpu as pltpu
from jax.experimental.pallas import tpu_sc as plsc

N = 10000
H = 128
OUT = 16
R = 3
E = 200000

NC = 2
NS = 16
TILES = NC * NS

K = 128
NBUF = 3
CPT = 51
E_PAD = CPT * TILES * K
PAD = E_PAD - E

WB = 80
NWB = N // WB
WPT = -(-NWB // NS)


def _make_sc_agg(h_dim, tab_rows):
    mesh = plsc.VectorSubcoreMesh(core_axis_name="c", subcore_axis_name="s")

    @functools.partial(
        pl.kernel,
        mesh=mesh,
        out_type=jax.ShapeDtypeStruct((NC, N, h_dim), jnp.float32),
        scratch_types=[
            pltpu.VMEM_SHARED((N + 8, h_dim), jnp.float32),
            pltpu.VMEM((NBUF, 2, K), jnp.int32),
            pltpu.VMEM((K, h_dim), jnp.float32),
            pltpu.VMEM((K, h_dim), jnp.float32),
            pltpu.VMEM((K, h_dim), jnp.float32),
            pltpu.SemaphoreType.DMA,
            pltpu.SemaphoreType.DMA,
            pltpu.SemaphoreType.DMA,
            pltpu.SemaphoreType.DMA,
            pltpu.SemaphoreType.DMA,
            pltpu.SemaphoreType.DMA,
        ],
    )
    def agg(tab, eall, zeros, out, acc, eidx, rows0, rows1, rows2,
            gs0, gs1, gs2, ss0, ss1, ss2):
        c = lax.axis_index("c")
        s = lax.axis_index("s")
        wid = c * NS + s
        bufs = ((rows0, gs0, ss0), (rows1, gs1, ss1), (rows2, gs2, ss2))

        pltpu.sync_copy(zeros, rows0.at[pl.ds(0, WB)])
        for k in range(WPT):
            m = s + NS * k

            @pl.when(m < NWB)
            def _(m=m):
                r0 = pl.multiple_of(m * WB, WB)
                pltpu.sync_copy(rows0.at[pl.ds(0, WB)], acc.at[pl.ds(r0, WB)])

        plsc.subcore_barrier()

        base0 = wid * CPT
        for r in range(R):
            roff = r * 2 * E_PAD

            def idx_copy(j, b, roff=roff):
                off = roff + (base0 + j) * K
                pltpu.sync_copy(eall.at[pl.ds(off, K)], eidx.at[b, 0])
                pltpu.sync_copy(eall.at[pl.ds(E_PAD + off, K)], eidx.at[b, 1])

            def gather(b, rb, sem):
                pltpu.async_copy(tab.at[eidx.at[b, 0]], rb, sem)

            def gather_wait(b, rb, sem):
                pltpu.make_async_copy(tab.at[eidx.at[b, 0]], rb, sem).wait()

            def scatter(b, rb, sem):
                pltpu.async_copy(rb, acc.at[eidx.at[b, 1]], sem, add=True)

            def scatter_wait(b, rb, sem):
                pltpu.make_async_copy(rb, acc.at[eidx.at[b, 1]], sem).wait()

            for b, (rb, gsem, ssem) in enumerate(bufs):
                idx_copy(b, b)
                gather(b, rb, gsem)

            def step(t, _):
                for b, (rb, gsem, ssem) in enumerate(bufs):
                    gather_wait(b, rb, gsem)
                    scatter(b, rb, ssem)
                for b, (rb, gsem, ssem) in enumerate(bufs):
                    scatter_wait(b, rb, ssem)
                    idx_copy(NBUF * t + NBUF + b, b)
                    gather(b, rb, gsem)
                return 0

            lax.fori_loop(0, CPT // NBUF - 1, step, 0)

            for b, (rb, gsem, ssem) in enumerate(bufs):
                gather_wait(b, rb, gsem)
                scatter(b, rb, ssem)
            for b, (rb, gsem, ssem) in enumerate(bufs):
                scatter_wait(b, rb, ssem)

        plsc.subcore_barrier()

        for k in range(WPT):
            m = s + NS * k

            @pl.when(m < NWB)
            def _(m=m):
                r0 = pl.multiple_of(m * WB, WB)
                pltpu.sync_copy(acc.at[pl.ds(r0, WB)], rows0.at[pl.ds(0, WB)])
                pltpu.sync_copy(rows0.at[pl.ds(0, WB)], out.at[c, pl.ds(r0, WB)])

    return agg


_sc_agg_1t = _make_sc_agg(H, N)
_sc_agg_3t = _make_sc_agg(H, R * N)


BN = 400


def _tc_dense_body(p_ref, b_ref, w_ref, o_ref):
    h = jnp.maximum(p_ref[0] + p_ref[1] + b_ref[0], 0.0)
    for r in range(R):
        o_ref[r] = jnp.dot(h, w_ref[r], preferred_element_type=jnp.float32)


def _tc_dense(part, b, w):
    return pl.pallas_call(
        _tc_dense_body,
        grid=(N // BN,),
        in_specs=[
            pl.BlockSpec((NC, BN, H), lambda i: (0, i, 0)),
            pl.BlockSpec((1, H), lambda i: (0, 0)),
            pl.BlockSpec((R, H, H), lambda i: (0, 0, 0)),
        ],
        out_specs=pl.BlockSpec((R, BN, H), lambda i: (0, i, 0)),
        out_shape=jax.ShapeDtypeStruct((R, N, H), jnp.float32),
    )(part, b, w)


def _tc_final_body(p_ref, b_ref, o_ref):
    o_ref[...] = p_ref[0, :, :OUT] + p_ref[1, :, :OUT] + b_ref[0]


def _tc_final(part, b):
    return pl.pallas_call(
        _tc_final_body,
        grid=(N // BN,),
        in_specs=[
            pl.BlockSpec((NC, BN, H), lambda i: (0, i, 0)),
            pl.BlockSpec((1, OUT), lambda i: (0, 0)),
        ],
        out_specs=pl.BlockSpec((BN, OUT), lambda i: (i, 0)),
        out_shape=jax.ShapeDtypeStruct((N, OUT), jnp.float32),
    )(part, b)


@jax.jit
def kernel(embed, b0, w1, b1, w2, b2, edge_index_0, edge_index_1,
           edge_index_2):
    zeros_h = jnp.zeros((WB, H), jnp.float32)
    w2_pad = jnp.zeros((R, H, H), jnp.float32).at[:, :, :OUT].set(w2)
    zpad = jnp.arange(PAD, dtype=jnp.int32) % N
    npad = N + (jnp.arange(PAD, dtype=jnp.int32) % 8)
    edges = (edge_index_0, edge_index_1, edge_index_2)
    eall_0 = jnp.concatenate(
        [jnp.concatenate([e[0], zpad, e[1], npad]) for e in edges])
    eall_s = jnp.concatenate(
        [jnp.concatenate([e[0] + r * N, zpad + r * N, e[1], npad])
         for r, e in enumerate(edges)])

    p0 = _sc_agg_1t(embed, eall_0, zeros_h)
    y = _tc_dense(p0, b0.reshape(1, H), w1)
    p1 = _sc_agg_3t(y.reshape(R * N, H), eall_s, zeros_h)
    z = _tc_dense(p1, b1.reshape(1, H), w2_pad)
    p2 = _sc_agg_3t(z.reshape(R * N, H), eall_s, zeros_h)
    return _tc_final(p2, b2.reshape(1, OUT))


# speedup vs baseline: 1.1846x; 1.0784x over previous
import functools

import jax
import jax.numpy as jnp
from jax import lax
from jax.experimental import pallas as pl
from jax.experimental.pallas import tpu as pltpu
from jax.experimental.pallas import tpu_sc as plsc

N = 10000
H = 128
OUT = 16
R = 3
E = 200000

NC = 2
NS = 16
TILES = NC * NS

K = 128
NBUF = 3
CPT = 51
E_PAD = CPT * TILES * K
PAD = E_PAD - E

WB = 80
NWB = N // WB
WPT = -(-NWB // NS)


def _make_sc_agg(h_dim, tab_rows):
    mesh = plsc.VectorSubcoreMesh(core_axis_name="c", subcore_axis_name="s")

    @functools.partial(
        pl.kernel,
        mesh=mesh,
        out_type=jax.ShapeDtypeStruct((NC, N, h_dim), jnp.float32),
        scratch_types=[
            pltpu.VMEM_SHARED((N + 8, h_dim), jnp.float32),
            pltpu.VMEM((NBUF, 2, K), jnp.int32),
            pltpu.VMEM((K, h_dim), jnp.float32),
            pltpu.VMEM((K, h_dim), jnp.float32),
            pltpu.VMEM((K, h_dim), jnp.float32),
            pltpu.SemaphoreType.DMA,
            pltpu.SemaphoreType.DMA,
            pltpu.SemaphoreType.DMA,
            pltpu.SemaphoreType.DMA,
            pltpu.SemaphoreType.DMA,
            pltpu.SemaphoreType.DMA,
        ],
    )
    def agg(tab, eall, zeros, out, acc, eidx, rows0, rows1, rows2,
            gs0, gs1, gs2, ss0, ss1, ss2):
        c = lax.axis_index("c")
        s = lax.axis_index("s")
        wid = c * NS + s
        bufs = ((rows0, gs0, ss0), (rows1, gs1, ss1), (rows2, gs2, ss2))

        pltpu.sync_copy(zeros, rows0.at[pl.ds(0, WB)])
        for k in range(WPT):
            m = s + NS * k

            @pl.when(m < NWB)
            def _(m=m):
                r0 = pl.multiple_of(m * WB, WB)
                pltpu.sync_copy(rows0.at[pl.ds(0, WB)], acc.at[pl.ds(r0, WB)])

        plsc.subcore_barrier()

        base0 = wid * CPT
        for r in range(R):
            roff = r * 2 * E_PAD

            def idx_copy(j, b, roff=roff):
                off = roff + (base0 + j) * K
                pltpu.sync_copy(eall.at[pl.ds(off, K)], eidx.at[b, 0])
                pltpu.sync_copy(eall.at[pl.ds(E_PAD + off, K)], eidx.at[b, 1])

            def gather(b, rb, sem):
                pltpu.async_copy(tab.at[eidx.at[b, 0]], rb, sem)

            def gather_wait(b, rb, sem):
                pltpu.make_async_copy(tab.at[eidx.at[b, 0]], rb, sem).wait()

            def scatter(b, rb, sem):
                pltpu.async_copy(rb, acc.at[eidx.at[b, 1]], sem, add=True)

            def scatter_wait(b, rb, sem):
                pltpu.make_async_copy(rb, acc.at[eidx.at[b, 1]], sem).wait()

            for b, (rb, gsem, ssem) in enumerate(bufs):
                idx_copy(b, b)
                gather(b, rb, gsem)

            def step(t, _):
                for b, (rb, gsem, ssem) in enumerate(bufs):
                    gather_wait(b, rb, gsem)
                    scatter(b, rb, ssem)
                for b, (rb, gsem, ssem) in enumerate(bufs):
                    scatter_wait(b, rb, ssem)
                    idx_copy(NBUF * t + NBUF + b, b)
                    gather(b, rb, gsem)
                return 0

            lax.fori_loop(0, CPT // NBUF - 1, step, 0)

            for b, (rb, gsem, ssem) in enumerate(bufs):
                gather_wait(b, rb, gsem)
                scatter(b, rb, ssem)
            for b, (rb, gsem, ssem) in enumerate(bufs):
                scatter_wait(b, rb, ssem)

        plsc.subcore_barrier()

        for k in range(WPT):
            m = s + NS * k

            @pl.when(m < NWB)
            def _(m=m):
                r0 = pl.multiple_of(m * WB, WB)
                pltpu.sync_copy(acc.at[pl.ds(r0, WB)], rows0.at[pl.ds(0, WB)])
                pltpu.sync_copy(rows0.at[pl.ds(0, WB)], out.at[c, pl.ds(r0, WB)])

    return agg


_sc_agg_1t = _make_sc_agg(H, N)
_sc_agg_3t = _make_sc_agg(H, R * N)


def _make_sc_agg16():
    mesh = plsc.VectorSubcoreMesh(core_axis_name="c", subcore_axis_name="s")
    TROWS = R * N
    TWB = 400
    NTWB = TROWS // TWB

    @functools.partial(
        pl.kernel,
        mesh=mesh,
        compiler_params=pltpu.CompilerParams(use_tc_tiling_on_sc=False),
        out_type=jax.ShapeDtypeStruct((NC, N, OUT), jnp.float32),
        scratch_types=[
            pltpu.VMEM_SHARED((TROWS, OUT), jnp.float32),
            pltpu.VMEM_SHARED((N + 8, OUT), jnp.float32),
            pltpu.VMEM((NBUF, 2, K), jnp.int32),
            pltpu.VMEM((K, OUT), jnp.float32),
            pltpu.VMEM((K, OUT), jnp.float32),
            pltpu.VMEM((K, OUT), jnp.float32),
            pltpu.VMEM((TWB, OUT), jnp.float32),
            pltpu.SemaphoreType.DMA,
            pltpu.SemaphoreType.DMA,
            pltpu.SemaphoreType.DMA,
            pltpu.SemaphoreType.DMA,
            pltpu.SemaphoreType.DMA,
            pltpu.SemaphoreType.DMA,
        ],
    )
    def agg16(tab_hbm, eall, zeros, out, tab, acc, eidx, rows0, rows1,
              rows2, sbuf, gs0, gs1, gs2, ss0, ss1, ss2):
        c = lax.axis_index("c")
        s = lax.axis_index("s")
        wid = c * NS + s
        bufs = ((rows0, gs0, ss0), (rows1, gs1, ss1), (rows2, gs2, ss2))

        for k in range(-(-NTWB // NS)):
            m = s + NS * k

            @pl.when(m < NTWB)
            def _(m=m):
                r0 = pl.multiple_of(m * TWB, TWB)
                pltpu.sync_copy(tab_hbm.at[pl.ds(r0, TWB)], sbuf)
                pltpu.sync_copy(sbuf, tab.at[pl.ds(r0, TWB)])

        pltpu.sync_copy(zeros, sbuf.at[pl.ds(0, WB)])
        for k in range(WPT):
            m = s + NS * k

            @pl.when(m < NWB)
            def _(m=m):
                r0 = pl.multiple_of(m * WB, WB)
                pltpu.sync_copy(sbuf.at[pl.ds(0, WB)], acc.at[pl.ds(r0, WB)])

        plsc.subcore_barrier()

        base0 = wid * CPT
        for r in range(R):
            roff = r * 2 * E_PAD

            def idx_copy(j, b, roff=roff):
                off = roff + (base0 + j) * K
                pltpu.sync_copy(eall.at[pl.ds(off, K)], eidx.at[b, 0])
                pltpu.sync_copy(eall.at[pl.ds(E_PAD + off, K)], eidx.at[b, 1])

            def gather(b, rb, sem):
                pltpu.async_copy(tab.at[eidx.at[b, 0]], rb, sem)

            def gather_wait(b, rb, sem):
                pltpu.make_async_copy(tab.at[eidx.at[b, 0]], rb, sem).wait()

            def scatter(b, rb, sem):
                pltpu.async_copy(rb, acc.at[eidx.at[b, 1]], sem, add=True)

            def scatter_wait(b, rb, sem):
                pltpu.make_async_copy(rb, acc.at[eidx.at[b, 1]], sem).wait()

            for b, (rb, gsem, ssem) in enumerate(bufs):
                idx_copy(b, b)
                gather(b, rb, gsem)

            def step(t, _):
                for b, (rb, gsem, ssem) in enumerate(bufs):
                    gather_wait(b, rb, gsem)
                    scatter(b, rb, ssem)
                for b, (rb, gsem, ssem) in enumerate(bufs):
                    scatter_wait(b, rb, ssem)
                    idx_copy(NBUF * t + NBUF + b, b)
                    gather(b, rb, gsem)
                return 0

            lax.fori_loop(0, CPT // NBUF - 1, step, 0)

            for b, (rb, gsem, ssem) in enumerate(bufs):
                gather_wait(b, rb, gsem)
                scatter(b, rb, ssem)
            for b, (rb, gsem, ssem) in enumerate(bufs):
                scatter_wait(b, rb, ssem)

        plsc.subcore_barrier()

        for k in range(WPT):
            m = s + NS * k

            @pl.when(m < NWB)
            def _(m=m):
                r0 = pl.multiple_of(m * WB, WB)
                pltpu.sync_copy(acc.at[pl.ds(r0, WB)], sbuf.at[pl.ds(0, WB)])
                pltpu.sync_copy(sbuf.at[pl.ds(0, WB)], out.at[c, pl.ds(r0, WB)])

    return agg16


_sc_agg16 = _make_sc_agg16()


BN = 400


def _tc_dense_body(p_ref, b_ref, w_ref, o_ref):
    h = jnp.maximum(p_ref[0] + p_ref[1] + b_ref[0], 0.0)
    for r in range(R):
        o_ref[r] = jnp.dot(h, w_ref[r], preferred_element_type=jnp.float32)


def _tc_dense(part, b, w, h_out):
    return pl.pallas_call(
        _tc_dense_body,
        grid=(N // BN,),
        in_specs=[
            pl.BlockSpec((NC, BN, H), lambda i: (0, i, 0)),
            pl.BlockSpec((1, H), lambda i: (0, 0)),
            pl.BlockSpec((R, H, h_out), lambda i: (0, 0, 0)),
        ],
        out_specs=pl.BlockSpec((R, BN, h_out), lambda i: (0, i, 0)),
        out_shape=jax.ShapeDtypeStruct((R, N, h_out), jnp.float32),
    )(part, b, w)


def _tc_final_body(p_ref, b_ref, o_ref):
    o_ref[...] = p_ref[0] + p_ref[1] + b_ref[0]


def _tc_final(part, b):
    return pl.pallas_call(
        _tc_final_body,
        grid=(N // BN,),
        in_specs=[
            pl.BlockSpec((NC, BN, OUT), lambda i: (0, i, 0)),
            pl.BlockSpec((1, OUT), lambda i: (0, 0)),
        ],
        out_specs=pl.BlockSpec((BN, OUT), lambda i: (i, 0)),
        out_shape=jax.ShapeDtypeStruct((N, OUT), jnp.float32),
    )(part, b)


@jax.jit
def kernel(embed, b0, w1, b1, w2, b2, edge_index_0, edge_index_1,
           edge_index_2):
    zeros_h = jnp.zeros((WB, H), jnp.float32)
    zeros_o = jnp.zeros((WB, OUT), jnp.float32)
    zpad = jnp.arange(PAD, dtype=jnp.int32) % N
    npad = N + (jnp.arange(PAD, dtype=jnp.int32) % 8)
    edges = (edge_index_0, edge_index_1, edge_index_2)
    eall_0 = jnp.concatenate(
        [jnp.concatenate([e[0], zpad, e[1], npad]) for e in edges])
    eall_s = jnp.concatenate(
        [jnp.concatenate([e[0] + r * N, zpad + r * N, e[1], npad])
         for r, e in enumerate(edges)])

    p0 = _sc_agg_1t(embed, eall_0, zeros_h)
    y = _tc_dense(p0, b0.reshape(1, H), w1, H)
    p1 = _sc_agg_3t(y.reshape(R * N, H), eall_s, zeros_h)
    z = _tc_dense(p1, b1.reshape(1, H), w2, OUT)
    p2 = _sc_agg16(z.reshape(R * N, OUT), eall_s, zeros_o)
    return _tc_final(p2, b2.reshape(1, OUT))


# speedup vs baseline: 1.3740x; 1.1599x over previous
import functools

import jax
import jax.numpy as jnp
from jax import lax
from jax.experimental import pallas as pl
from jax.experimental.pallas import tpu as pltpu
from jax.experimental.pallas import tpu_sc as plsc

N = 10000
H = 128
OUT = 16
R = 3
E = 200000

NC = 2
NS = 16
TILES = NC * NS

K = 128
NBUF = 3
CPT = 51
E_PAD = CPT * TILES * K
PAD = E_PAD - E

WB = 80
NWB = N // WB
WPT = -(-NWB // NS)


def _make_sc_agg(h_dim, tab_rows):
    mesh = plsc.VectorSubcoreMesh(core_axis_name="c", subcore_axis_name="s")

    @functools.partial(
        pl.kernel,
        mesh=mesh,
        out_type=jax.ShapeDtypeStruct((NC, N, h_dim), jnp.float32),
        scratch_types=[
            pltpu.VMEM_SHARED((N + 8, h_dim), jnp.float32),
            pltpu.VMEM((NBUF, 2, K), jnp.int32),
            pltpu.VMEM((K, h_dim), jnp.float32),
            pltpu.VMEM((K, h_dim), jnp.float32),
            pltpu.VMEM((K, h_dim), jnp.float32),
            pltpu.SemaphoreType.DMA,
            pltpu.SemaphoreType.DMA,
            pltpu.SemaphoreType.DMA,
            pltpu.SemaphoreType.DMA,
            pltpu.SemaphoreType.DMA,
            pltpu.SemaphoreType.DMA,
        ],
    )
    def agg(tab, eall, zeros, out, acc, eidx, rows0, rows1, rows2,
            gs0, gs1, gs2, ss0, ss1, ss2):
        c = lax.axis_index("c")
        s = lax.axis_index("s")
        wid = c * NS + s
        bufs = ((rows0, gs0, ss0), (rows1, gs1, ss1), (rows2, gs2, ss2))

        pltpu.sync_copy(zeros, rows0.at[pl.ds(0, WB)])
        for k in range(WPT):
            m = s + NS * k

            @pl.when(m < NWB)
            def _(m=m):
                r0 = pl.multiple_of(m * WB, WB)
                pltpu.sync_copy(rows0.at[pl.ds(0, WB)], acc.at[pl.ds(r0, WB)])

        plsc.subcore_barrier()

        base0 = wid * CPT
        for r in range(R):
            roff = r * 2 * E_PAD

            def idx_copy(j, b, roff=roff):
                off = roff + (base0 + j) * K
                pltpu.sync_copy(eall.at[pl.ds(off, K)], eidx.at[b, 0])
                pltpu.sync_copy(eall.at[pl.ds(E_PAD + off, K)], eidx.at[b, 1])

            def gather(b, rb, sem):
                pltpu.async_copy(tab.at[eidx.at[b, 0]], rb, sem)

            def gather_wait(b, rb, sem):
                pltpu.make_async_copy(tab.at[eidx.at[b, 0]], rb, sem).wait()

            def scatter(b, rb, sem):
                pltpu.async_copy(rb, acc.at[eidx.at[b, 1]], sem, add=True)

            def scatter_wait(b, rb, sem):
                pltpu.make_async_copy(rb, acc.at[eidx.at[b, 1]], sem).wait()

            for b, (rb, gsem, ssem) in enumerate(bufs):
                idx_copy(b, b)
                gather(b, rb, gsem)

            def step(t, _):
                for b, (rb, gsem, ssem) in enumerate(bufs):
                    gather_wait(b, rb, gsem)
                    scatter(b, rb, ssem)
                for b, (rb, gsem, ssem) in enumerate(bufs):
                    scatter_wait(b, rb, ssem)
                    idx_copy(NBUF * t + NBUF + b, b)
                    gather(b, rb, gsem)
                return 0

            lax.fori_loop(0, CPT // NBUF - 1, step, 0)

            for b, (rb, gsem, ssem) in enumerate(bufs):
                gather_wait(b, rb, gsem)
                scatter(b, rb, ssem)
            for b, (rb, gsem, ssem) in enumerate(bufs):
                scatter_wait(b, rb, ssem)

        plsc.subcore_barrier()

        for k in range(WPT):
            m = s + NS * k

            @pl.when(m < NWB)
            def _(m=m):
                r0 = pl.multiple_of(m * WB, WB)
                pltpu.sync_copy(acc.at[pl.ds(r0, WB)], rows0.at[pl.ds(0, WB)])
                pltpu.sync_copy(rows0.at[pl.ds(0, WB)], out.at[c, pl.ds(r0, WB)])

    return agg


_sc_agg_1t = _make_sc_agg(H, N)
_sc_agg_3t = _make_sc_agg(H, R * N)


def _make_sc_agg16():
    mesh = plsc.VectorSubcoreMesh(core_axis_name="c", subcore_axis_name="s")
    TROWS = R * N
    TWB = 400
    NTWB = TROWS // TWB
    GT = CPT * TILES

    @functools.partial(
        pl.kernel,
        mesh=mesh,
        compiler_params=pltpu.CompilerParams(use_tc_tiling_on_sc=False),
        out_type=jax.ShapeDtypeStruct((NC, N, OUT), jnp.float32),
        scratch_types=[
            pltpu.VMEM_SHARED((TROWS, OUT), jnp.float32),
            pltpu.VMEM_SHARED((N + 8, OUT), jnp.float32),
            pltpu.VMEM((2, CPT, K), jnp.int32),
            pltpu.VMEM((K, OUT), jnp.float32),
            pltpu.VMEM((K, OUT), jnp.float32),
            pltpu.VMEM((K, OUT), jnp.float32),
            pltpu.VMEM((TWB, OUT), jnp.float32),
            pltpu.SemaphoreType.DMA,
            pltpu.SemaphoreType.DMA,
            pltpu.SemaphoreType.DMA,
            pltpu.SemaphoreType.DMA,
            pltpu.SemaphoreType.DMA,
            pltpu.SemaphoreType.DMA,
        ],
    )
    def agg16(tab_hbm, eall, zeros, out, tab, acc, eidx, rows0, rows1,
              rows2, sbuf, gs0, gs1, gs2, ss0, ss1, ss2):
        c = lax.axis_index("c")
        s = lax.axis_index("s")
        wid = c * NS + s
        bufs = ((rows0, gs0, ss0), (rows1, gs1, ss1), (rows2, gs2, ss2))

        for k in range(-(-NTWB // NS)):
            m = s + NS * k

            @pl.when(m < NTWB)
            def _(m=m):
                r0 = pl.multiple_of(m * TWB, TWB)
                pltpu.sync_copy(tab_hbm.at[pl.ds(r0, TWB)], sbuf)
                pltpu.sync_copy(sbuf, tab.at[pl.ds(r0, TWB)])

        pltpu.sync_copy(zeros, sbuf.at[pl.ds(0, WB)])
        for k in range(WPT):
            m = s + NS * k

            @pl.when(m < NWB)
            def _(m=m):
                r0 = pl.multiple_of(m * WB, WB)
                pltpu.sync_copy(sbuf.at[pl.ds(0, WB)], acc.at[pl.ds(r0, WB)])

        plsc.subcore_barrier()

        base0 = wid * CPT
        for r in range(R):
            pltpu.sync_copy(eall.at[pl.ds(2 * r * GT + base0, CPT)],
                            eidx.at[0])
            pltpu.sync_copy(eall.at[pl.ds((2 * r + 1) * GT + base0, CPT)],
                            eidx.at[1])

            def gather(j, rb, sem):
                pltpu.async_copy(tab.at[eidx.at[0, j]], rb, sem)

            def gather_wait(j, rb, sem):
                pltpu.make_async_copy(tab.at[eidx.at[0, j]], rb, sem).wait()

            def scatter(j, rb, sem):
                pltpu.async_copy(rb, acc.at[eidx.at[1, j]], sem, add=True)

            def scatter_wait(j, rb, sem):
                pltpu.make_async_copy(rb, acc.at[eidx.at[1, j]], sem).wait()

            for b, (rb, gsem, ssem) in enumerate(bufs):
                gather(b, rb, gsem)

            def step(t, _):
                for b, (rb, gsem, ssem) in enumerate(bufs):
                    gather_wait(NBUF * t + b, rb, gsem)
                    scatter(NBUF * t + b, rb, ssem)
                for b, (rb, gsem, ssem) in enumerate(bufs):
                    scatter_wait(NBUF * t + b, rb, ssem)
                    gather(NBUF * t + NBUF + b, rb, gsem)
                return 0

            lax.fori_loop(0, CPT // NBUF - 1, step, 0)

            last = CPT - NBUF
            for b, (rb, gsem, ssem) in enumerate(bufs):
                gather_wait(last + b, rb, gsem)
                scatter(last + b, rb, ssem)
            for b, (rb, gsem, ssem) in enumerate(bufs):
                scatter_wait(last + b, rb, ssem)

        plsc.subcore_barrier()

        for k in range(WPT):
            m = s + NS * k

            @pl.when(m < NWB)
            def _(m=m):
                r0 = pl.multiple_of(m * WB, WB)
                pltpu.sync_copy(acc.at[pl.ds(r0, WB)], sbuf.at[pl.ds(0, WB)])
                pltpu.sync_copy(sbuf.at[pl.ds(0, WB)], out.at[c, pl.ds(r0, WB)])

    return agg16


_sc_agg16 = _make_sc_agg16()


BN = 400


def _tc_dense_body(p_ref, b_ref, w_ref, o_ref):
    h = jnp.maximum(p_ref[0] + p_ref[1] + b_ref[0], 0.0)
    for r in range(R):
        o_ref[r] = jnp.dot(h, w_ref[r], preferred_element_type=jnp.float32)


def _tc_dense(part, b, w, h_out):
    return pl.pallas_call(
        _tc_dense_body,
        grid=(N // BN,),
        in_specs=[
            pl.BlockSpec((NC, BN, H), lambda i: (0, i, 0)),
            pl.BlockSpec((1, H), lambda i: (0, 0)),
            pl.BlockSpec((R, H, h_out), lambda i: (0, 0, 0)),
        ],
        out_specs=pl.BlockSpec((R, BN, h_out), lambda i: (0, i, 0)),
        out_shape=jax.ShapeDtypeStruct((R, N, h_out), jnp.float32),
    )(part, b, w)


def _tc_final_body(p_ref, b_ref, o_ref):
    o_ref[...] = p_ref[0] + p_ref[1] + b_ref[0]


def _tc_final(part, b):
    return pl.pallas_call(
        _tc_final_body,
        grid=(N // BN,),
        in_specs=[
            pl.BlockSpec((NC, BN, OUT), lambda i: (0, i, 0)),
            pl.BlockSpec((1, OUT), lambda i: (0, 0)),
        ],
        out_specs=pl.BlockSpec((BN, OUT), lambda i: (i, 0)),
        out_shape=jax.ShapeDtypeStruct((N, OUT), jnp.float32),
    )(part, b)


@jax.jit
def kernel(embed, b0, w1, b1, w2, b2, edge_index_0, edge_index_1,
           edge_index_2):
    zeros_h = jnp.zeros((WB, H), jnp.float32)
    zeros_o = jnp.zeros((WB, OUT), jnp.float32)
    zpad = jnp.arange(PAD, dtype=jnp.int32) % N
    npad = N + (jnp.arange(PAD, dtype=jnp.int32) % 8)
    edges = (edge_index_0, edge_index_1, edge_index_2)
    eall_0 = jnp.concatenate(
        [jnp.concatenate([e[0], zpad, e[1], npad]) for e in edges])
    eall_s = jnp.concatenate(
        [jnp.concatenate([e[0] + r * N, zpad + r * N, e[1], npad])
         for r, e in enumerate(edges)])

    p0 = _sc_agg_1t(embed, eall_0, zeros_h)
    y = _tc_dense(p0, b0.reshape(1, H), w1, H)
    p1 = _sc_agg_3t(y.reshape(R * N, H), eall_s, zeros_h)
    z = _tc_dense(p1, b1.reshape(1, H), w2, OUT)
    p2 = _sc_agg16(z.reshape(R * N, OUT), eall_s.reshape(-1, K), zeros_o)
    return _tc_final(p2, b2.reshape(1, OUT))
